# baseline (speedup 1.0000x reference)
import functools

import jax
import jax.numpy as jnp
from jax import lax
from jax.experimental import pallas as pl
from jax.experimental.pallas import tpu as pltpu
from jax.experimental.pallas import tpu_sc as plsc

NC = 2
NS = 16
NW = NC * NS
LANES = 16

CB = 512
PAD = CB + 1


@jax.jit
def _embed_add(xt, idx_lin, tblp):
    s_len, d, b_len = xt.shape
    n_bchunk = b_len // CB
    n_chunks = s_len * n_bchunk
    per_w = n_chunks // NW
    mesh = plsc.VectorSubcoreMesh(core_axis_name="c", subcore_axis_name="s")

    @functools.partial(
        pl.kernel,
        out_type=jax.ShapeDtypeStruct((s_len, d, b_len), jnp.float32),
        mesh=mesh,
        compiler_params=pltpu.CompilerParams(needs_layout_passes=False),
        scratch_types=[
            pltpu.VMEM((CB,), jnp.int32),
            pltpu.VMEM((CB, 2 * d), jnp.float32),
            pltpu.VMEM((d, PAD), jnp.float32),
            pltpu.SemaphoreType.DMA,
        ],
    )
    def main(x_hbm, idx_hbm, tbl_hbm, out_hbm, idx_v, gbuf, acc, sem):
        wid = lax.axis_index("s") * NC + lax.axis_index("c")
        lane = lax.iota(jnp.int32, LANES)

        def chunk(k, carry):
            g = wid * per_w + k
            si = g // n_bchunk
            b0 = (g % n_bchunk) * CB
            pltpu.sync_copy(idx_hbm.at[pl.ds(si * b_len + b0, CB)], idx_v)
            gat = pltpu.async_copy(tbl_hbm.at[idx_v], gbuf, sem)
            pltpu.sync_copy(x_hbm.at[si, :, pl.ds(b0, CB)], acc.at[:, pl.ds(0, CB)])
            gat.wait()

            def row(r, c2):
                rv = jnp.full((LANES,), r, jnp.int32)
                for kk in range(d // LANES):
                    v = gbuf[r, pl.ds(kk * LANES, LANES)]
                    plsc.addupdate_scatter(acc, [lane + kk * LANES, rv], v)
                return c2

            lax.fori_loop(0, CB, row, 0, unroll=4)
            pltpu.sync_copy(acc.at[:, pl.ds(0, CB)], out_hbm.at[si, :, pl.ds(b0, CB)])
            return carry

        lax.fori_loop(0, per_w, chunk, 0)

    return main(xt, idx_lin, tblp)


def kernel(x, in_chan_matrix, embed_weight):
    b, l, d = x.shape
    xt = jnp.transpose(x, (1, 2, 0))
    idx_lin = in_chan_matrix.astype(jnp.int32).T.reshape(b * l)
    tblp = jnp.pad(embed_weight, ((0, 0), (0, d)))
    ot = _embed_add(xt, idx_lin, tblp)
    return jnp.transpose(ot, (2, 0, 1))


# speedup vs baseline: 1.1228x; 1.1228x over previous
import functools

import jax
import jax.numpy as jnp
from jax import lax
from jax.experimental import pallas as pl
from jax.experimental.pallas import tpu as pltpu
from jax.experimental.pallas import tpu_sc as plsc

NC = 2
NS = 16
NW = NC * NS
LANES = 16

CB = 512
PAD = CB + 1


@jax.jit
def _embed_add(xt, idx_lin, tblp):
    s_len, d, b_len = xt.shape
    n_bchunk = b_len // CB
    n_chunks = s_len * n_bchunk
    per_w = n_chunks // NW
    mesh = plsc.VectorSubcoreMesh(core_axis_name="c", subcore_axis_name="s")

    @functools.partial(
        pl.kernel,
        out_type=jax.ShapeDtypeStruct((s_len, d, b_len), jnp.float32),
        mesh=mesh,
        compiler_params=pltpu.CompilerParams(needs_layout_passes=False),
        scratch_types=[
            pltpu.VMEM((CB,), jnp.int32),
            pltpu.VMEM((CB, 2 * d), jnp.float32),
            pltpu.VMEM((d, PAD), jnp.float32),
            pltpu.SemaphoreType.DMA,
        ],
    )
    def main(x_hbm, idx_hbm, tbl_hbm, out_hbm, idx_v, gbuf, acc, sem):
        wid = lax.axis_index("s") * NC + lax.axis_index("c")
        lane = lax.iota(jnp.int32, LANES)

        def chunk(k, carry):
            g = wid * per_w + k
            si = g // n_bchunk
            b0 = (g % n_bchunk) * CB
            pltpu.sync_copy(idx_hbm.at[pl.ds(si * b_len + b0, CB)], idx_v)
            gat = pltpu.async_copy(tbl_hbm.at[idx_v], gbuf, sem)
            pltpu.sync_copy(x_hbm.at[si, :, pl.ds(b0, CB)], acc.at[:, pl.ds(0, CB)])
            gat.wait()

            @plsc.parallel_loop(0, CB, unroll=4)
            def row(r):
                rv = jnp.full((LANES,), r, jnp.int32)
                for kk in range(d // LANES):
                    v = gbuf[r, pl.ds(kk * LANES, LANES)]
                    plsc.addupdate_scatter(acc, [lane + kk * LANES, rv], v)
            pltpu.sync_copy(acc.at[:, pl.ds(0, CB)], out_hbm.at[si, :, pl.ds(b0, CB)])
            return carry

        lax.fori_loop(0, per_w, chunk, 0)

    return main(xt, idx_lin, tblp)


def kernel(x, in_chan_matrix, embed_weight):
    b, l, d = x.shape
    xt = jnp.transpose(x, (1, 2, 0))
    idx_lin = in_chan_matrix.astype(jnp.int32).T.reshape(b * l)
    tblp = jnp.pad(embed_weight, ((0, 0), (0, d)))
    ot = _embed_add(xt, idx_lin, tblp)
    return jnp.transpose(ot, (2, 0, 1))


# speedup vs baseline: 1.1336x; 1.0097x over previous
import functools

import jax
import jax.numpy as jnp
from jax import lax
from jax.experimental import pallas as pl
from jax.experimental.pallas import tpu as pltpu
from jax.experimental.pallas import tpu_sc as plsc

NC = 2
NS = 16
NW = NC * NS

P = 8
CB = 400
BT = 512


def _sc_gather(piece, s_piece, b_len, n_total):
    rows = s_piece * b_len
    per_w = rows // NW
    n_chunks = per_w // CB
    mesh = plsc.VectorSubcoreMesh(core_axis_name="c", subcore_axis_name="s")

    @functools.partial(
        pl.kernel,
        out_type=jax.ShapeDtypeStruct((rows, 128), jnp.float32),
        mesh=mesh,
        compiler_params=pltpu.CompilerParams(needs_layout_passes=False),
        scratch_types=[
            pltpu.VMEM((CB,), jnp.int32),
            pltpu.VMEM((CB, 128), jnp.float32),
            pltpu.SemaphoreType.DMA,
        ],
    )
    def gath(idx_hbm, tbl_hbm, g_hbm, idx_v, gbuf, sem):
        wid = lax.axis_index("s") * NC + lax.axis_index("c")
        base = piece * rows + wid * per_w

        def chunk(k, carry):
            off = k * CB
            pltpu.sync_copy(idx_hbm.at[pl.ds(base + off, CB)], idx_v)
            pltpu.async_copy(tbl_hbm.at[idx_v], gbuf, sem).wait()
            pltpu.sync_copy(gbuf, g_hbm.at[pl.ds(wid * per_w + off, CB)])
            return carry

        lax.fori_loop(0, n_chunks, chunk, 0)

    return gath


def _tc_add(piece, s_piece, d, b_len):
    def body(g_ref, x_ref, o_ref):
        g = g_ref[...]
        t = jnp.transpose(g[:, :d])
        o_ref[...] = x_ref[...] + t[None, :, :]

    return pl.pallas_call(
        body,
        grid=(s_piece, b_len // BT),
        in_specs=[
            pl.BlockSpec((BT, 128), lambda i, j: (i * (b_len // BT) + j, 0)),
            pl.BlockSpec((1, d, BT), lambda i, j: (piece * s_piece + i, 0, j)),
        ],
        out_specs=pl.BlockSpec((1, d, BT), lambda i, j: (i, 0, j)),
        out_shape=jax.ShapeDtypeStruct((s_piece, d, b_len), jnp.float32),
    )


@jax.jit
def _embed_add(xt, idx_lin, tblp):
    s_len, d, b_len = xt.shape
    n_total = s_len * b_len
    s_piece = s_len // P
    pieces = []
    for p in range(P):
        g_p = _sc_gather(p, s_piece, b_len, n_total)(idx_lin, tblp)
        pieces.append(_tc_add(p, s_piece, d, b_len)(g_p, xt))
    return jnp.concatenate(pieces, axis=0)


def kernel(x, in_chan_matrix, embed_weight):
    b, l, d = x.shape
    xt = jnp.transpose(x, (1, 2, 0))
    idx_lin = in_chan_matrix.astype(jnp.int32).T.reshape(b * l)
    tblp = jnp.pad(embed_weight, ((0, 0), (0, d)))
    ot = _embed_add(xt, idx_lin, tblp)
    return jnp.transpose(ot, (2, 0, 1))


# speedup vs baseline: 1.6725x; 1.4754x over previous
import functools

import jax
import jax.numpy as jnp
from jax import lax
from jax.experimental import pallas as pl
from jax.experimental.pallas import tpu as pltpu
from jax.experimental.pallas import tpu_sc as plsc

NC = 2
NS = 16
NW = NC * NS
LANES = 16

P = 8
CB = 400
BT = 2048


def _sc_gather(piece, s_piece, b_len):
    rows = s_piece * b_len
    per_w = rows // NW
    n_chunks = per_w // CB
    mesh = plsc.VectorSubcoreMesh(core_axis_name="c", subcore_axis_name="s")

    @functools.partial(
        pl.kernel,
        out_type=jax.ShapeDtypeStruct((rows, 128), jnp.float32),
        mesh=mesh,
        compiler_params=pltpu.CompilerParams(
            use_tc_tiling_on_sc=False, needs_layout_passes=False
        ),
        scratch_types=[
            pltpu.VMEM((CB,), jnp.int32),
            pltpu.VMEM((CB,), jnp.int32),
            pltpu.VMEM((CB, 128), jnp.float32),
            pltpu.SemaphoreType.DMA,
        ],
    )
    def gath(idx_hbm, tbl_hbm, g_hbm, idx_v, gidx_v, gbuf, sem):
        wid = lax.axis_index("s") * NC + lax.axis_index("c")
        base = piece * rows + wid * per_w

        def chunk(k, carry):
            off = k * CB
            pltpu.sync_copy(idx_hbm.at[pl.ds(base + off, CB)], idx_v)

            def prep(j, c2):
                sl = pl.ds(j * LANES, LANES)
                gidx_v[sl] = lax.shift_right_logical(idx_v[sl], 1)
                return c2

            lax.fori_loop(0, CB // LANES, prep, 0, unroll=4)
            pltpu.async_copy(tbl_hbm.at[gidx_v], gbuf, sem).wait()
            pltpu.sync_copy(gbuf, g_hbm.at[pl.ds(wid * per_w + off, CB)])
            return carry

        lax.fori_loop(0, n_chunks, chunk, 0)

    return gath


def _tc_add(piece, s_piece, s_len, d, b_len, aliased):
    nb = b_len // BT

    def body(*refs):
        if aliased:
            _, g_ref, i_ref, x_ref, o_ref = refs
        else:
            g_ref, i_ref, x_ref, o_ref = refs
        t = jnp.transpose(g_ref[...])
        h = (i_ref[...] & 1) != 0
        o_ref[...] = x_ref[...] + jnp.where(h[None, :], t[d:, :], t[:d, :])

    in_specs = [
        pl.BlockSpec((BT, 128), lambda i, j: (i * nb + j, 0)),
        pl.BlockSpec((BT,), lambda i, j: (piece * s_piece * nb + i * nb + j,)),
        pl.BlockSpec((None, d, BT), lambda i, j: (piece * s_piece + i, 0, j)),
    ]
    kwargs = {}
    if aliased:
        in_specs = [pl.BlockSpec(memory_space=pl.ANY)] + in_specs
        kwargs["input_output_aliases"] = {0: 0}
    return pl.pallas_call(
        body,
        grid=(s_piece, nb),
        in_specs=in_specs,
        out_specs=pl.BlockSpec((None, d, BT), lambda i, j: (piece * s_piece + i, 0, j)),
        out_shape=jax.ShapeDtypeStruct((s_len, d, b_len), jnp.float32),
        **kwargs,
    )


@jax.jit
def _embed_add(xt, idx_lin, tbl2):
    s_len, d, b_len = xt.shape
    s_piece = s_len // P
    gs = [_sc_gather(p, s_piece, b_len)(idx_lin, tbl2) for p in range(P)]
    out = _tc_add(0, s_piece, s_len, d, b_len, False)(gs[0], idx_lin, xt)
    for p in range(1, P):
        out = _tc_add(p, s_piece, s_len, d, b_len, True)(out, gs[p], idx_lin, xt)
    return out


def kernel(x, in_chan_matrix, embed_weight):
    b, l, d = x.shape
    v = embed_weight.shape[0]
    xt = jnp.transpose(x, (1, 2, 0))
    idx_lin = in_chan_matrix.astype(jnp.int32).T.reshape(b * l)
    tbl2 = embed_weight.reshape(v // 2, 2 * d)
    ot = _embed_add(xt, idx_lin, tbl2)
    return jnp.transpose(ot, (2, 0, 1))


# speedup vs baseline: 1.7136x; 1.0246x over previous
import functools

import jax
import jax.numpy as jnp
from jax import lax
from jax.experimental import pallas as pl
from jax.experimental.pallas import tpu as pltpu
from jax.experimental.pallas import tpu_sc as plsc

NC = 2
NS = 16
NW = NC * NS
LANES = 16

P = 8
CB = 400
BT = 2048


def _sc_gather(piece, s_piece, b_len):
    rows = s_piece * b_len
    per_w = rows // NW
    n_chunks = per_w // CB
    mesh = plsc.VectorSubcoreMesh(core_axis_name="c", subcore_axis_name="s")

    @functools.partial(
        pl.kernel,
        out_type=jax.ShapeDtypeStruct((rows, 64), jnp.float32),
        mesh=mesh,
        compiler_params=pltpu.CompilerParams(
            use_tc_tiling_on_sc=False, needs_layout_passes=False
        ),
        scratch_types=[
            pltpu.VMEM((CB,), jnp.int32),
            pltpu.VMEM((CB, 64), jnp.float32),
            pltpu.SemaphoreType.DMA,
        ],
    )
    def gath(idx_hbm, tbl_hbm, g_hbm, idx_v, gbuf, sem):
        wid = lax.axis_index("s") * NC + lax.axis_index("c")
        base = piece * rows + wid * per_w

        def chunk(k, carry):
            off = k * CB
            pltpu.sync_copy(idx_hbm.at[pl.ds(base + off, CB)], idx_v)
            pltpu.async_copy(tbl_hbm.at[idx_v], gbuf, sem).wait()
            pltpu.sync_copy(gbuf, g_hbm.at[pl.ds(wid * per_w + off, CB)])
            return carry

        lax.fori_loop(0, n_chunks, chunk, 0)

    return gath


def _tc_add(piece, s_piece, s_len, d, b_len, aliased):
    nb = b_len // BT

    def body(*refs):
        if aliased:
            _, g_ref, x_ref, o_ref = refs
        else:
            g_ref, x_ref, o_ref = refs
        t = jnp.transpose(g_ref[...])
        o_ref[...] = x_ref[...] + jnp.concatenate([t[:d, :], t[d:, :]], axis=1)

    in_specs = [
        pl.BlockSpec((BT // 2, 128), lambda i, j: (i * nb + j, 0)),
        pl.BlockSpec((None, d, BT), lambda i, j: (piece * s_piece + i, 0, j)),
    ]
    kwargs = {}
    if aliased:
        in_specs = [pl.BlockSpec(memory_space=pl.ANY)] + in_specs
        kwargs["input_output_aliases"] = {0: 0}
    return pl.pallas_call(
        body,
        grid=(s_piece, nb),
        in_specs=in_specs,
        out_specs=pl.BlockSpec((None, d, BT), lambda i, j: (piece * s_piece + i, 0, j)),
        out_shape=jax.ShapeDtypeStruct((s_len, d, b_len), jnp.float32),
        **kwargs,
    )


@jax.jit
def _embed_add(xt, idx_lin, tbl2):
    s_len, d, b_len = xt.shape
    s_piece = s_len // P
    gs = [_sc_gather(p, s_piece, b_len)(idx_lin, tbl2) for p in range(P)]
    g2 = [g.reshape(g.shape[0] // 2, 128) for g in gs]
    out = _tc_add(0, s_piece, s_len, d, b_len, False)(g2[0], xt)
    for p in range(1, P):
        out = _tc_add(p, s_piece, s_len, d, b_len, True)(out, g2[p], xt)
    return out


def kernel(x, in_chan_matrix, embed_weight):
    b, l, d = x.shape
    v = embed_weight.shape[0]
    xt = jnp.transpose(x, (1, 2, 0))
    idx_lin = in_chan_matrix.astype(jnp.int32).T.reshape(b * l)
    idxp = (
        idx_lin.reshape(-1, 2, BT // 2).transpose(0, 2, 1).reshape(b * l)
    )
    ot = _embed_add(xt, idxp, embed_weight)
    return jnp.transpose(ot, (2, 0, 1))


# speedup vs baseline: 1.7851x; 1.0417x over previous
import functools

import jax
import jax.numpy as jnp
from jax import lax
from jax.experimental import pallas as pl
from jax.experimental.pallas import tpu as pltpu
from jax.experimental.pallas import tpu_sc as plsc

NC = 2
NS = 16
NW = NC * NS
LANES = 16

P = 8
CB = 400
BT = 2048


def _sc_gather(piece, s_piece, b_len):
    rows = s_piece * b_len
    per_w = rows // NW
    n_chunks = per_w // CB
    mesh = plsc.VectorSubcoreMesh(core_axis_name="c", subcore_axis_name="s")

    @functools.partial(
        pl.kernel,
        out_type=jax.ShapeDtypeStruct((rows, 128), jnp.float32),
        mesh=mesh,
        compiler_params=pltpu.CompilerParams(
            use_tc_tiling_on_sc=False, needs_layout_passes=False
        ),
        scratch_types=[
            pltpu.VMEM((CB,), jnp.int32),
            pltpu.VMEM((CB, 128), jnp.float32),
            pltpu.SemaphoreType.DMA,
        ],
    )
    def gath(idx_hbm, tbl_hbm, g_hbm, idx_v, gbuf, sem):
        wid = lax.axis_index("s") * NC + lax.axis_index("c")
        base = piece * rows + wid * per_w

        def chunk(k, carry):
            off = k * CB
            pltpu.sync_copy(idx_hbm.at[pl.ds(base + off, CB)], idx_v)
            pltpu.async_copy(tbl_hbm.at[idx_v], gbuf, sem).wait()
            pltpu.sync_copy(gbuf, g_hbm.at[pl.ds(wid * per_w + off, CB)])
            return carry

        lax.fori_loop(0, n_chunks, chunk, 0)

    return gath


def _tc_add(piece, s_piece, s_len, d, b_len, aliased):
    nb = b_len // BT

    def body(*refs):
        if aliased:
            _, g_ref, x_ref, o_ref = refs
        else:
            g_ref, x_ref, o_ref = refs
        t = jnp.transpose(g_ref[...])
        o_ref[...] = x_ref[...] + t[:d, :]

    in_specs = [
        pl.BlockSpec((BT, 128), lambda i, j: (i * nb + j, 0)),
        pl.BlockSpec((None, d, BT), lambda i, j: (piece * s_piece + i, 0, j)),
    ]
    kwargs = {}
    if aliased:
        in_specs = [pl.BlockSpec(memory_space=pl.ANY)] + in_specs
        kwargs["input_output_aliases"] = {0: 0}
    return pl.pallas_call(
        body,
        grid=(s_piece, nb),
        in_specs=in_specs,
        out_specs=pl.BlockSpec((None, d, BT), lambda i, j: (piece * s_piece + i, 0, j)),
        out_shape=jax.ShapeDtypeStruct((s_len, d, b_len), jnp.float32),
        **kwargs,
    )


@jax.jit
def _embed_add(xt, idx_lin, tbl2):
    s_len, d, b_len = xt.shape
    s_piece = s_len // P
    gs = [_sc_gather(p, s_piece, b_len)(idx_lin, tbl2) for p in range(P)]
    out = _tc_add(0, s_piece, s_len, d, b_len, False)(gs[0], xt)
    for p in range(1, P):
        out = _tc_add(p, s_piece, s_len, d, b_len, True)(out, gs[p], xt)
    return out


def kernel(x, in_chan_matrix, embed_weight):
    b, l, d = x.shape
    v = embed_weight.shape[0]
    xt = jnp.transpose(x, (1, 2, 0))
    idx_lin = in_chan_matrix.astype(jnp.int32).T.reshape(b * l)
    tblp = jnp.pad(embed_weight, ((0, 0), (0, d)))
    ot = _embed_add(xt, idx_lin, tblp)
    return jnp.transpose(ot, (2, 0, 1))
